# baseline (speedup 1.0000x reference)
import functools

import jax
import jax.numpy as jnp
from jax import lax
from jax.experimental import pallas as pl
from jax.experimental.pallas import tpu as pltpu
from jax.experimental.pallas import tpu_sc as plsc

B = 8
C = 80
H = 128
W = 128
HW = H * W
K_STATIC = 100
KPAD = 128
CB = 4
CBLK = C // CB
NROWS = 104
NPAD = 128


def _pair_argmax(v, p):
    r = v.shape[0]
    while r > 1:
        h = r // 2
        va, vb = v[:h], v[h:]
        pa, pb = p[:h], p[h:]
        m = (va > vb) | ((va == vb) & (pa < pb))
        v = jnp.where(m, va, vb)
        p = jnp.where(m, pa, pb)
        r = h
    w = v.shape[1]
    while w > 1:
        h = w // 2
        va, vb = v[:, :h], v[:, h:]
        pa, pb = p[:, :h], p[:, h:]
        m = (va > vb) | ((va == vb) & (pa < pb))
        v = jnp.where(m, va, vb)
        p = jnp.where(m, pa, pb)
        w = h
    return v, p


def _nms_topk_body(fmap_ref, scores_ref, inds_ref,
                   masked_ref, rowmax_ref, cand_ref, posm_ref):
    b = pl.program_id(0)
    cb = pl.program_id(1)
    x = fmap_ref[0]
    neg = jnp.float32(-jnp.inf)
    left = jnp.concatenate([x[:, :, 1:], jnp.full((CBLK, H, 1), neg)], axis=2)
    right = jnp.concatenate([jnp.full((CBLK, H, 1), neg), x[:, :, :-1]], axis=2)
    mw = jnp.maximum(jnp.maximum(left, right), x)
    up = jnp.concatenate([mw[:, 1:, :], jnp.full((CBLK, 1, W), neg)], axis=1)
    down = jnp.concatenate([jnp.full((CBLK, 1, W), neg), mw[:, :-1, :]], axis=1)
    m9 = jnp.maximum(jnp.maximum(up, down), mw)
    masked = jnp.where(m9 == x, x, jnp.float32(0.0))
    masked_ref[pl.ds(cb * CBLK * H, CBLK * H), :] = masked.reshape(CBLK * H, W)
    rowmax_ref[pl.ds(cb * CBLK, CBLK), :] = masked.max(axis=2)

    @pl.when(cb == CB - 1)
    def _extract():
        lane = lax.broadcasted_iota(jnp.int32, (1, W), 1)
        neg1 = jnp.float32(-1.0)
        rpos = (lax.broadcasted_iota(jnp.int32, (NPAD, H), 0) * H
                + lax.broadcasted_iota(jnp.int32, (NPAD, H), 1))

        rm0 = jnp.concatenate(
            [rowmax_ref[...], jnp.full((NPAD - C, H), neg1)], axis=0)

        def rowsel(i, carry):
            rm, racc = carry
            m, p = _pair_argmax(rm, rpos)
            racc = jnp.where(lane == i, p, racc)
            rm = jnp.where(rpos == p, neg1, rm)
            return rm, racc

        _, rowids = lax.fori_loop(
            0, NROWS, rowsel, (rm0, jnp.zeros((1, W), jnp.int32)))

        for k in range(NROWS):
            p_k = jnp.sum(jnp.where(lane == k, rowids, jnp.int32(0)))
            cand_ref[pl.ds(k, 1), :] = masked_ref[pl.ds(p_k, 1), :]
            posm_ref[pl.ds(k, 1), :] = p_k * W + lane
        padpos = ((lax.broadcasted_iota(jnp.int32, (NPAD - NROWS, W), 0)
                   + C * H + NROWS) * W
                  + lax.broadcasted_iota(jnp.int32, (NPAD - NROWS, W), 1))
        cand_ref[pl.ds(NROWS, NPAD - NROWS), :] = jnp.full(
            (NPAD - NROWS, W), neg1)
        posm_ref[pl.ds(NROWS, NPAD - NROWS), :] = padpos

        posm = posm_ref[...]

        def body(i, carry):
            cand, sacc, iacc = carry
            m, ind = _pair_argmax(cand, posm)
            sacc = jnp.where(lane == i, m, sacc)
            iacc = jnp.where(lane == i, ind, iacc)
            cand = jnp.where(posm == ind, neg1, cand)
            return cand, sacc, iacc

        _, sacc, iacc = lax.fori_loop(
            0, K_STATIC, body,
            (cand_ref[...], jnp.zeros((1, W), jnp.float32),
             jnp.zeros((1, W), jnp.int32)))
        scores_ref[...] = sacc.reshape(1, 1, KPAD)
        inds_ref[...] = iacc.reshape(1, 1, KPAD)


def _nms_topk(fmap):
    return pl.pallas_call(
        _nms_topk_body,
        grid=(B, CB),
        in_specs=[pl.BlockSpec((1, CBLK, H, W), lambda b, cb: (b, cb, 0, 0))],
        out_specs=[
            pl.BlockSpec((1, 1, KPAD), lambda b, cb: (b, 0, 0)),
            pl.BlockSpec((1, 1, KPAD), lambda b, cb: (b, 0, 0)),
        ],
        out_shape=[
            jax.ShapeDtypeStruct((B, 1, KPAD), jnp.float32),
            jax.ShapeDtypeStruct((B, 1, KPAD), jnp.int32),
        ],
        scratch_shapes=[
            pltpu.VMEM((C * H, W), jnp.float32),
            pltpu.VMEM((C, H), jnp.float32),
            pltpu.VMEM((NPAD, W), jnp.float32),
            pltpu.VMEM((NPAD, W), jnp.int32),
        ],
    )(fmap)


def _sc_decode_body(whf, regf, indsf, x1o, y1o, x2o, y2o, clso,
                    inds_v, idxa_v, idxb_v, whx_v, why_v, rgx_v, rgy_v,
                    x1_v, y1_v, x2_v, y2_v, cls_v,
                    sem0, sem1, sem2, sem3):
    wid = lax.axis_index("s") * 2 + lax.axis_index("c")

    @pl.when(wid < B)
    def _():
        b = wid
        base = b * (2 * HW)
        pltpu.sync_copy(indsf.at[pl.ds(b * KPAD, KPAD)], inds_v)
        for j in range(KPAD // 16):
            sl = pl.ds(j * 16, 16)
            sp = lax.rem(inds_v[sl], jnp.int32(HW))
            idxa_v[sl] = sp + base
            idxb_v[sl] = sp + (base + HW)
        c0 = pltpu.async_copy(whf.at[idxa_v], whx_v, sem0)
        c1 = pltpu.async_copy(whf.at[idxb_v], why_v, sem1)
        c2 = pltpu.async_copy(regf.at[idxa_v], rgx_v, sem2)
        c3 = pltpu.async_copy(regf.at[idxb_v], rgy_v, sem3)
        c0.wait()
        c1.wait()
        c2.wait()
        c3.wait()
        for j in range(KPAD // 16):
            sl = pl.ds(j * 16, 16)
            ind = inds_v[sl]
            sp = lax.rem(ind, jnp.int32(HW))
            cls_v[sl] = lax.convert_element_type(
                lax.div(ind, jnp.int32(HW)), jnp.float32)
            ys = lax.convert_element_type(
                lax.div(sp, jnp.int32(W)), jnp.float32) + rgy_v[sl]
            xs = lax.convert_element_type(
                lax.rem(sp, jnp.int32(W)), jnp.float32) + rgx_v[sl]
            hw2 = whx_v[sl] * jnp.float32(0.5)
            hh2 = why_v[sl] * jnp.float32(0.5)
            x1_v[sl] = xs - hw2
            y1_v[sl] = ys - hh2
            x2_v[sl] = xs + hw2
            y2_v[sl] = ys + hh2
        pltpu.sync_copy(x1_v, x1o.at[pl.ds(b * KPAD, KPAD)])
        pltpu.sync_copy(y1_v, y1o.at[pl.ds(b * KPAD, KPAD)])
        pltpu.sync_copy(x2_v, x2o.at[pl.ds(b * KPAD, KPAD)])
        pltpu.sync_copy(y2_v, y2o.at[pl.ds(b * KPAD, KPAD)])
        pltpu.sync_copy(cls_v, clso.at[pl.ds(b * KPAD, KPAD)])


def _sc_decode(wh_flat, reg_flat, inds_flat):
    f32 = jnp.float32
    fn = pl.kernel(
        _sc_decode_body,
        mesh=plsc.VectorSubcoreMesh(core_axis_name="c", subcore_axis_name="s"),
        out_type=[jax.ShapeDtypeStruct((B * KPAD,), f32)] * 5,
        scratch_types=(
            [pltpu.VMEM((KPAD,), jnp.int32)] * 3
            + [pltpu.VMEM((KPAD,), f32)] * 9
            + [pltpu.SemaphoreType.DMA] * 4
        ),
    )
    return fn(wh_flat, reg_flat, inds_flat)


def kernel(fmap, wh, reg, K):
    scores, inds = _nms_topk(fmap)
    scores = scores.reshape(B, KPAD)
    x1, y1, x2, y2, cls = _sc_decode(
        wh.reshape(B * 2 * HW), reg.reshape(B * 2 * HW),
        inds.reshape(B * KPAD))
    x1, y1, x2, y2, cls = (a.reshape(B, KPAD) for a in (x1, y1, x2, y2, cls))
    k_zero = jnp.asarray(K, jnp.float32) - jnp.float32(K_STATIC)
    bboxes = jnp.stack([x1, y1, x2, y2], axis=2)[:, :K_STATIC, :]
    scores_out = scores[:, :K_STATIC, None] + k_zero
    clses = cls[:, :K_STATIC, None]
    return bboxes, scores_out, clses


# speedup vs baseline: 4.3404x; 4.3404x over previous
import functools

import jax
import jax.numpy as jnp
from jax import lax
from jax.experimental import pallas as pl
from jax.experimental.pallas import tpu as pltpu
from jax.experimental.pallas import tpu_sc as plsc

B = 8
C = 80
H = 128
W = 128
HW = H * W
K_STATIC = 100
KPAD = 128
CB = 4
CBLK = C // CB
NROWS = 104
NPAD = 128


def _bitonic_desc(v, p):
    rows = lax.broadcasted_iota(jnp.int32, (NPAD, W), 0)
    lanes = lax.broadcasted_iota(jnp.int32, (NPAD, W), 1)
    n = NPAD * W
    size = 2
    while size <= n:
        d = size // 2
        while d >= 1:
            if d < W:
                vm = pltpu.roll(v, W - d, axis=1)
                vp = pltpu.roll(v, d, axis=1)
                pm = pltpu.roll(p, W - d, axis=1)
                pp = pltpu.roll(p, d, axis=1)
                upper = (lanes & d) == 0
            else:
                dr = d // W
                vm = pltpu.roll(v, NPAD - dr, axis=0)
                vp = pltpu.roll(v, dr, axis=0)
                pm = pltpu.roll(p, NPAD - dr, axis=0)
                pp = pltpu.roll(p, dr, axis=0)
                upper = (rows & dr) == 0
            pv = jnp.where(upper, vm, vp)
            pq = jnp.where(upper, pm, pp)
            self_wins = (v > pv) | ((v == pv) & (p < pq))
            if size < W:
                dirdesc = (lanes & size) == 0
            else:
                dirdesc = (rows & (size // W)) == 0
            take = self_wins == (upper == dirdesc)
            v = jnp.where(take, v, pv)
            p = jnp.where(take, p, pq)
            d //= 2
        size *= 2
    return v, p


def _nms_topk_body(fmap_ref, scores_ref, inds_ref,
                   masked_ref, rowmax_ref, cand_ref, posm_ref):
    b = pl.program_id(0)
    cb = pl.program_id(1)
    x = fmap_ref[0]
    neg = jnp.float32(-jnp.inf)
    left = jnp.concatenate([x[:, :, 1:], jnp.full((CBLK, H, 1), neg)], axis=2)
    right = jnp.concatenate([jnp.full((CBLK, H, 1), neg), x[:, :, :-1]], axis=2)
    mw = jnp.maximum(jnp.maximum(left, right), x)
    up = jnp.concatenate([mw[:, 1:, :], jnp.full((CBLK, 1, W), neg)], axis=1)
    down = jnp.concatenate([jnp.full((CBLK, 1, W), neg), mw[:, :-1, :]], axis=1)
    m9 = jnp.maximum(jnp.maximum(up, down), mw)
    masked = jnp.where(m9 == x, x, jnp.float32(0.0))
    masked_ref[pl.ds(cb * CBLK * H, CBLK * H), :] = masked.reshape(CBLK * H, W)
    rowmax_ref[pl.ds(cb * CBLK, CBLK), :] = masked.max(axis=2)

    @pl.when(cb == CB - 1)
    def _extract():
        lane = lax.broadcasted_iota(jnp.int32, (1, W), 1)
        neg1 = jnp.float32(-1.0)
        rpos = (lax.broadcasted_iota(jnp.int32, (NPAD, H), 0) * H
                + lax.broadcasted_iota(jnp.int32, (NPAD, H), 1))

        rm0 = jnp.concatenate(
            [rowmax_ref[...], jnp.full((NPAD - C, H), neg1)], axis=0)
        _, rps = _bitonic_desc(rm0, rpos)
        rowids = rps[0:1, :]

        for k in range(NPAD):
            p_k = jnp.sum(jnp.where(lane == k, rowids, jnp.int32(0)))
            cand_ref[pl.ds(k, 1), :] = masked_ref[pl.ds(p_k, 1), :]
            posm_ref[pl.ds(k, 1), :] = p_k * W + lane

        vs, ps = _bitonic_desc(cand_ref[...], posm_ref[...])
        scores_ref[...] = vs[0:1, :].reshape(1, 1, KPAD)
        inds_ref[...] = ps[0:1, :].reshape(1, 1, KPAD)


def _nms_topk(fmap):
    return pl.pallas_call(
        _nms_topk_body,
        grid=(B, CB),
        in_specs=[pl.BlockSpec((1, CBLK, H, W), lambda b, cb: (b, cb, 0, 0))],
        out_specs=[
            pl.BlockSpec((1, 1, KPAD), lambda b, cb: (b, 0, 0)),
            pl.BlockSpec((1, 1, KPAD), lambda b, cb: (b, 0, 0)),
        ],
        out_shape=[
            jax.ShapeDtypeStruct((B, 1, KPAD), jnp.float32),
            jax.ShapeDtypeStruct((B, 1, KPAD), jnp.int32),
        ],
        scratch_shapes=[
            pltpu.VMEM((C * H, W), jnp.float32),
            pltpu.VMEM((C, H), jnp.float32),
            pltpu.VMEM((NPAD, W), jnp.float32),
            pltpu.VMEM((NPAD, W), jnp.int32),
        ],
    )(fmap)


def _sc_decode_body(whf, regf, indsf, x1o, y1o, x2o, y2o, clso,
                    inds_v, idxa_v, idxb_v, whx_v, why_v, rgx_v, rgy_v,
                    x1_v, y1_v, x2_v, y2_v, cls_v,
                    sem0, sem1, sem2, sem3):
    wid = lax.axis_index("s") * 2 + lax.axis_index("c")

    @pl.when(wid < B)
    def _():
        b = wid
        base = b * (2 * HW)
        pltpu.sync_copy(indsf.at[pl.ds(b * KPAD, KPAD)], inds_v)
        for j in range(KPAD // 16):
            sl = pl.ds(j * 16, 16)
            sp = lax.rem(inds_v[sl], jnp.int32(HW))
            idxa_v[sl] = sp + base
            idxb_v[sl] = sp + (base + HW)
        c0 = pltpu.async_copy(whf.at[idxa_v], whx_v, sem0)
        c1 = pltpu.async_copy(whf.at[idxb_v], why_v, sem1)
        c2 = pltpu.async_copy(regf.at[idxa_v], rgx_v, sem2)
        c3 = pltpu.async_copy(regf.at[idxb_v], rgy_v, sem3)
        c0.wait()
        c1.wait()
        c2.wait()
        c3.wait()
        for j in range(KPAD // 16):
            sl = pl.ds(j * 16, 16)
            ind = inds_v[sl]
            sp = lax.rem(ind, jnp.int32(HW))
            cls_v[sl] = lax.convert_element_type(
                lax.div(ind, jnp.int32(HW)), jnp.float32)
            ys = lax.convert_element_type(
                lax.div(sp, jnp.int32(W)), jnp.float32) + rgy_v[sl]
            xs = lax.convert_element_type(
                lax.rem(sp, jnp.int32(W)), jnp.float32) + rgx_v[sl]
            hw2 = whx_v[sl] * jnp.float32(0.5)
            hh2 = why_v[sl] * jnp.float32(0.5)
            x1_v[sl] = xs - hw2
            y1_v[sl] = ys - hh2
            x2_v[sl] = xs + hw2
            y2_v[sl] = ys + hh2
        pltpu.sync_copy(x1_v, x1o.at[pl.ds(b * KPAD, KPAD)])
        pltpu.sync_copy(y1_v, y1o.at[pl.ds(b * KPAD, KPAD)])
        pltpu.sync_copy(x2_v, x2o.at[pl.ds(b * KPAD, KPAD)])
        pltpu.sync_copy(y2_v, y2o.at[pl.ds(b * KPAD, KPAD)])
        pltpu.sync_copy(cls_v, clso.at[pl.ds(b * KPAD, KPAD)])


def _sc_decode(wh_flat, reg_flat, inds_flat):
    f32 = jnp.float32
    fn = pl.kernel(
        _sc_decode_body,
        mesh=plsc.VectorSubcoreMesh(core_axis_name="c", subcore_axis_name="s"),
        out_type=[jax.ShapeDtypeStruct((B * KPAD,), f32)] * 5,
        scratch_types=(
            [pltpu.VMEM((KPAD,), jnp.int32)] * 3
            + [pltpu.VMEM((KPAD,), f32)] * 9
            + [pltpu.SemaphoreType.DMA] * 4
        ),
    )
    return fn(wh_flat, reg_flat, inds_flat)


def kernel(fmap, wh, reg, K):
    scores, inds = _nms_topk(fmap)
    scores = scores.reshape(B, KPAD)
    x1, y1, x2, y2, cls = _sc_decode(
        wh.reshape(B * 2 * HW), reg.reshape(B * 2 * HW),
        inds.reshape(B * KPAD))
    x1, y1, x2, y2, cls = (a.reshape(B, KPAD) for a in (x1, y1, x2, y2, cls))
    k_zero = jnp.asarray(K, jnp.float32) - jnp.float32(K_STATIC)
    bboxes = jnp.stack([x1, y1, x2, y2], axis=2)[:, :K_STATIC, :]
    scores_out = scores[:, :K_STATIC, None] + k_zero
    clses = cls[:, :K_STATIC, None]
    return bboxes, scores_out, clses
